# baseline (speedup 1.0000x reference)
import functools

import jax
import jax.numpy as jnp
from jax.experimental import pallas as pl
from jax.experimental.pallas import tpu as pltpu

_B = 64
_BB = 64


def _fused_body(t_ref, sac_ref, somac_ref, x_ref, n_ref, o_ref):
    i = pl.program_id(0)
    for r in range(_BB):
        ti = t_ref[i * _BB + r]
        c1 = sac_ref[ti]
        c2 = somac_ref[ti]
        o_ref[r] = c1 * x_ref[r] + c2 * n_ref[r]


@jax.jit
def _tc_fused(t, sac, somac, x, n):
    blk = (_BB,) + x.shape[1:]
    imap = lambda i, *_: (i, 0, 0, 0)
    grid_spec = pltpu.PrefetchScalarGridSpec(
        num_scalar_prefetch=3,
        grid=(_B // _BB,),
        in_specs=[
            pl.BlockSpec(blk, imap),
            pl.BlockSpec(blk, imap),
        ],
        out_specs=pl.BlockSpec(blk, imap),
    )
    return pl.pallas_call(
        _fused_body,
        grid_spec=grid_spec,
        out_shape=jax.ShapeDtypeStruct(x.shape, jnp.float32),
        compiler_params=pltpu.CompilerParams(
            dimension_semantics=("parallel",)),
    )(t, sac, somac, x, n)


def kernel(x_start, noise, sqrt_alphas_cumprod, sqrt_one_minus_alphas_cumprod, t):
    return _tc_fused(t, sqrt_alphas_cumprod, sqrt_one_minus_alphas_cumprod,
                     x_start, noise)


# speedup vs baseline: 1.0715x; 1.0715x over previous
import functools

import jax
import jax.numpy as jnp
from jax.experimental import pallas as pl
from jax.experimental.pallas import tpu as pltpu

_B = 64
_BB = 32


def _fused_body(t_ref, sac_ref, somac_ref, x_ref, n_ref, o_ref):
    i = pl.program_id(0)
    for r in range(_BB):
        ti = t_ref[i * _BB + r]
        c1 = sac_ref[ti]
        c2 = somac_ref[ti]
        o_ref[r] = c1 * x_ref[r] + c2 * n_ref[r]


@jax.jit
def _tc_fused(t, sac, somac, x, n):
    blk = (_BB,) + x.shape[1:]
    imap = lambda i, *_: (i, 0, 0, 0)
    grid_spec = pltpu.PrefetchScalarGridSpec(
        num_scalar_prefetch=3,
        grid=(_B // _BB,),
        in_specs=[
            pl.BlockSpec(blk, imap),
            pl.BlockSpec(blk, imap),
        ],
        out_specs=pl.BlockSpec(blk, imap),
    )
    return pl.pallas_call(
        _fused_body,
        grid_spec=grid_spec,
        out_shape=jax.ShapeDtypeStruct(x.shape, jnp.float32),
        compiler_params=pltpu.CompilerParams(
            dimension_semantics=("parallel",)),
    )(t, sac, somac, x, n)


def kernel(x_start, noise, sqrt_alphas_cumprod, sqrt_one_minus_alphas_cumprod, t):
    return _tc_fused(t, sqrt_alphas_cumprod, sqrt_one_minus_alphas_cumprod,
                     x_start, noise)
